# baseline (speedup 1.0000x reference)
import functools

import jax
import jax.numpy as jnp
from jax import lax
from jax.experimental import pallas as pl
from jax.experimental.pallas import tpu as pltpu
from jax.experimental.pallas import tpu_sc as plsc

N = 10000
E = 320000
F = 128
UNITS = 64
C_OUT = 40
D = 48
DEG_COL = 40

NW = 32
CH = 128
NCH = 80
EPW = NCH * CH
E_PAD = NW * EPW
N_ACC = 10240
STRIPE = N_ACC // 16
ROWBLK = 1000
BATCH = 5
NSUP = NCH // BATCH


def _prep_body(x_ref, w_ref, b_ref, w2_ref, b2_ref, y0_ref, t0_ref, u_ref):
    w_out = w2_ref[...]
    a0 = jnp.dot(w_ref[0] - w_ref[2], w_out, preferred_element_type=jnp.float32)
    a1 = -jnp.dot(w_ref[1], w_out, preferred_element_type=jnp.float32)
    a2 = 2.0 * jnp.dot(w_ref[2], w_out, preferred_element_type=jnp.float32)
    pad = jnp.zeros((F, D - C_OUT), jnp.float32)
    xb = x_ref[...]
    bias = jnp.dot(b_ref[...], w_out, preferred_element_type=jnp.float32) + b2_ref[...]
    biasp = jnp.concatenate([bias, jnp.zeros((1, D - C_OUT), jnp.float32)], axis=1)
    y0_ref[...] = jnp.dot(xb, jnp.concatenate([a0, pad], axis=1),
                          preferred_element_type=jnp.float32) + biasp
    t0_ref[...] = jnp.dot(xb, jnp.concatenate([a1, pad], axis=1),
                          preferred_element_type=jnp.float32)
    ids = lax.broadcasted_iota(jnp.int32, (1, D), 1)
    one_col = jnp.where(ids == DEG_COL, 1.0, 0.0)
    u_ref[...] = jnp.dot(xb, jnp.concatenate([a2, pad], axis=1),
                         preferred_element_type=jnp.float32) + one_col


def _prep(x, W, b2d, W2, b22d):
    grid = (N // ROWBLK,)
    out_shape = [jax.ShapeDtypeStruct((N, D), jnp.float32)] * 3
    return pl.pallas_call(
        _prep_body,
        grid=grid,
        in_specs=[
            pl.BlockSpec((ROWBLK, F), lambda i: (i, 0)),
            pl.BlockSpec((3, F, UNITS), lambda i: (0, 0, 0)),
            pl.BlockSpec((1, UNITS), lambda i: (0, 0)),
            pl.BlockSpec((UNITS, C_OUT), lambda i: (0, 0)),
            pl.BlockSpec((1, C_OUT), lambda i: (0, 0)),
        ],
        out_specs=[pl.BlockSpec((ROWBLK, D), lambda i: (i, 0))] * 3,
        out_shape=out_shape,
    )(x, W, b2d, W2, b22d)


def _sc_pass_body(col_hbm, row_hbm, ew_hbm, table_hbm, out_hbm,
                  colv, rowv, ewv, rows_a, rows_b, acc, sem_a, sem_b):
    cid = lax.axis_index("c")
    sid = lax.axis_index("s")
    wid = sid * 2 + cid

    pltpu.sync_copy(col_hbm.at[wid], colv)
    pltpu.sync_copy(row_hbm.at[wid], rowv)
    pltpu.sync_copy(ew_hbm.at[pl.ds(wid * EPW, EPW)], ewv)

    zero16 = jnp.zeros((16,), jnp.float32)
    za = rows_a.at[0]

    def _zrow(e, carry):
        for k in range(D // 16):
            za[e, pl.ds(16 * k, 16)] = zero16
        return carry

    lax.fori_loop(0, CH, _zrow, 0)
    for i in range(STRIPE // CH):
        pltpu.sync_copy(za, acc.at[pl.ds(sid * STRIPE + i * CH, CH)])
    plsc.subcore_barrier()

    def _fire(s, buf, sem):
        for k in range(BATCH):
            pltpu.async_copy(table_hbm.at[colv.at[s * BATCH + k]],
                             buf.at[k], sem)

    def _drain(buf, sem):
        for k in range(BATCH):
            pltpu.make_async_copy(table_hbm.at[colv.at[0]],
                                  buf.at[k], sem).wait()

    def _process(s, buf):
        for k in range(BATCH):
            j = s * BATCH + k
            bk = buf.at[k]

            def _scale(g, carry2, j=j, bk=bk):
                ew16 = ewv[pl.ds(j * CH + g * 16, 16)]
                for e in range(16):
                    splat = ew16.at[jnp.full((16,), e, jnp.int32)].get(
                        mode="promise_in_bounds")
                    r = g * 16 + e
                    for q in range(D // 16):
                        bk[r, pl.ds(16 * q, 16)] = bk[r, pl.ds(16 * q, 16)] * splat
                return carry2

            lax.fori_loop(0, CH // 16, _scale, 0)
            pltpu.sync_copy(bk, acc.at[rowv.at[j]], add=True)

    _fire(0, rows_a, sem_a)

    def _pair(g, carry):
        s0 = 2 * g
        _drain(rows_a, sem_a)
        _fire(s0 + 1, rows_b, sem_b)
        _process(s0, rows_a)
        _drain(rows_b, sem_b)
        _fire(lax.rem(s0 + 2, NSUP), rows_a, sem_a)
        _process(s0 + 1, rows_b)
        return carry

    lax.fori_loop(0, NSUP // 2, _pair, 0)
    _drain(rows_a, sem_a)
    plsc.subcore_barrier()

    for i in range(STRIPE // CH):
        off = sid * STRIPE + i * CH
        pltpu.sync_copy(acc.at[pl.ds(off, CH)], za)
        pltpu.sync_copy(za, out_hbm.at[cid, pl.ds(off, CH)])


_sc_pass = pl.kernel(
    _sc_pass_body,
    out_type=jax.ShapeDtypeStruct((2, N_ACC, D), jnp.float32),
    mesh=plsc.VectorSubcoreMesh(core_axis_name="c", subcore_axis_name="s"),
    scratch_types=[
        pltpu.VMEM((NCH, CH), jnp.int32),
        pltpu.VMEM((NCH, CH), jnp.int32),
        pltpu.VMEM((EPW,), jnp.float32),
        pltpu.VMEM((BATCH, CH, D), jnp.float32),
        pltpu.VMEM((BATCH, CH, D), jnp.float32),
        pltpu.VMEM_SHARED((N_ACC, D), jnp.float32),
        pltpu.SemaphoreType.DMA,
        pltpu.SemaphoreType.DMA,
    ],
    compiler_params=pltpu.CompilerParams(use_tc_tiling_on_sc=False),
)


def _mid_body(acc_ref, t0_ref, taug_ref, dinv_ref):
    s = acc_ref[0] + acc_ref[1]
    deg = s[:, DEG_COL:DEG_COL + 1]
    dinv = 1.0 / jnp.where(deg > 0, deg, 1.0)
    taug_ref[...] = t0_ref[...] + s * dinv
    dinv_ref[...] = dinv


def _mid(acc1, t0):
    grid = (N // ROWBLK,)
    return pl.pallas_call(
        _mid_body,
        grid=grid,
        in_specs=[
            pl.BlockSpec((2, ROWBLK, D), lambda i: (0, i, 0)),
            pl.BlockSpec((ROWBLK, D), lambda i: (i, 0)),
        ],
        out_specs=[
            pl.BlockSpec((ROWBLK, D), lambda i: (i, 0)),
            pl.BlockSpec((ROWBLK, 1), lambda i: (i, 0)),
        ],
        out_shape=[
            jax.ShapeDtypeStruct((N, D), jnp.float32),
            jax.ShapeDtypeStruct((N, 1), jnp.float32),
        ],
    )(acc1, t0)


def _final_body(acc_ref, y0_ref, dinv_ref, out_ref):
    s = acc_ref[0] + acc_ref[1]
    out_ref[...] = y0_ref[:, :C_OUT] + s[:, :C_OUT] * dinv_ref[...]


def _final(acc2, y0b, dinv):
    grid = (N // ROWBLK,)
    return pl.pallas_call(
        _final_body,
        grid=grid,
        in_specs=[
            pl.BlockSpec((2, ROWBLK, D), lambda i: (0, i, 0)),
            pl.BlockSpec((ROWBLK, D), lambda i: (i, 0)),
            pl.BlockSpec((ROWBLK, 1), lambda i: (i, 0)),
        ],
        out_specs=pl.BlockSpec((ROWBLK, C_OUT), lambda i: (i, 0)),
        out_shape=jax.ShapeDtypeStruct((N, C_OUT), jnp.float32),
    )(acc2, y0b, dinv)


def kernel(x, edge_index, edge_weight, W, b, W2, b2):
    pad = (0, E_PAD - E)
    row3 = jnp.pad(edge_index[0], pad).reshape(NW, NCH, CH)
    col3 = jnp.pad(edge_index[1], pad).reshape(NW, NCH, CH)
    ew_p = jnp.pad(edge_weight, pad)
    b2d = b.reshape(1, UNITS)
    b22d = b2.reshape(1, C_OUT)

    y0b, t0, u_aug = _prep(x, W, b2d, W2, b22d)
    acc1 = _sc_pass(col3, row3, ew_p, u_aug)
    t_aug, dinv = _mid(acc1, t0)
    acc2 = _sc_pass(col3, row3, ew_p, t_aug)
    return _final(acc2, y0b, dinv)


# speedup vs baseline: 1.2308x; 1.2308x over previous
import jax
import jax.numpy as jnp
from jax import lax
from jax.experimental import pallas as pl
from jax.experimental.pallas import tpu as pltpu
from jax.experimental.pallas import tpu_sc as plsc

N = 10000
E = 320000
F = 128
UNITS = 64
C_OUT = 40
D = 48
DT = 32
DEG_COL = 40

NW = 32
CH = 128
NCH = 80
EPW = NCH * CH
E_PAD = NW * EPW
N_ACC = 10240
STRIPE = N_ACC // 16
ROWBLK = 1000
BATCH = 4
NSUP = NCH // BATCH


def _rnbf(x):
    b = lax.bitcast_convert_type(x, jnp.int32)
    rnd = b + jnp.int32(0x7FFF) + (lax.shift_right_arithmetic(b, 16) & 1)
    return lax.shift_right_logical(rnd, 16)


def _to_table(val48):
    g0 = jnp.left_shift(_rnbf(val48[:, 16:32]), 16) | _rnbf(val48[:, 0:16])
    g1 = _rnbf(val48[:, 32:48])
    return jnp.concatenate([g0, g1], axis=1)


def _prep_body(x_ref, w_ref, b_ref, w2_ref, b2_ref, y0_ref, t0_ref, u_ref):
    w_out = w2_ref[...]
    a0 = jnp.dot(w_ref[0] - w_ref[2], w_out, preferred_element_type=jnp.float32)
    a1 = -jnp.dot(w_ref[1], w_out, preferred_element_type=jnp.float32)
    a2 = 2.0 * jnp.dot(w_ref[2], w_out, preferred_element_type=jnp.float32)
    pad = jnp.zeros((F, D - C_OUT), jnp.float32)
    xb = x_ref[...]
    bias = jnp.dot(b_ref[...], w_out, preferred_element_type=jnp.float32) + b2_ref[...]
    biasp = jnp.concatenate([bias, jnp.zeros((1, D - C_OUT), jnp.float32)], axis=1)
    y0_ref[...] = jnp.dot(xb, jnp.concatenate([a0, pad], axis=1),
                          preferred_element_type=jnp.float32) + biasp
    t0_ref[...] = jnp.dot(xb, jnp.concatenate([a1, pad], axis=1),
                          preferred_element_type=jnp.float32)
    ids = lax.broadcasted_iota(jnp.int32, (1, D), 1)
    one_col = jnp.where(ids == DEG_COL, 1.0, 0.0)
    u48 = jnp.dot(xb, jnp.concatenate([a2, pad], axis=1),
                  preferred_element_type=jnp.float32) + one_col
    u_ref[...] = _to_table(u48)


def _prep(x, W, b2d, W2, b22d):
    grid = (N // ROWBLK,)
    return pl.pallas_call(
        _prep_body,
        grid=grid,
        in_specs=[
            pl.BlockSpec((ROWBLK, F), lambda i: (i, 0)),
            pl.BlockSpec((3, F, UNITS), lambda i: (0, 0, 0)),
            pl.BlockSpec((1, UNITS), lambda i: (0, 0)),
            pl.BlockSpec((UNITS, C_OUT), lambda i: (0, 0)),
            pl.BlockSpec((1, C_OUT), lambda i: (0, 0)),
        ],
        out_specs=[
            pl.BlockSpec((ROWBLK, D), lambda i: (i, 0)),
            pl.BlockSpec((ROWBLK, D), lambda i: (i, 0)),
            pl.BlockSpec((ROWBLK, DT), lambda i: (i, 0)),
        ],
        out_shape=[
            jax.ShapeDtypeStruct((N, D), jnp.float32),
            jax.ShapeDtypeStruct((N, D), jnp.float32),
            jax.ShapeDtypeStruct((N, DT), jnp.int32),
        ],
    )(x, W, b2d, W2, b22d)


def _sc_pass_body(col_hbm, row_hbm, ew_hbm, table_hbm, out_hbm,
                  colv, rowv, ewv, bf_a, bf_b, fbuf, acc, sem_a, sem_b):
    cid = lax.axis_index("c")
    sid = lax.axis_index("s")
    wid = sid * 2 + cid

    pltpu.sync_copy(col_hbm.at[wid], colv)
    pltpu.sync_copy(row_hbm.at[wid], rowv)
    pltpu.sync_copy(ew_hbm.at[pl.ds(wid * EPW, EPW)], ewv)

    zero16 = jnp.zeros((16,), jnp.float32)

    def _zrow(e, carry):
        for k in range(D // 16):
            fbuf[e, pl.ds(16 * k, 16)] = zero16
        return carry

    lax.fori_loop(0, CH, _zrow, 0)
    for i in range(STRIPE // CH):
        pltpu.sync_copy(fbuf, acc.at[pl.ds(sid * STRIPE + i * CH, CH)])
    plsc.subcore_barrier()

    def _fire(s, buf, sem):
        for k in range(BATCH):
            pltpu.async_copy(table_hbm.at[colv.at[s * BATCH + k]],
                             buf.at[k], sem)

    def _drain(buf, sem):
        for k in range(BATCH):
            pltpu.make_async_copy(table_hbm.at[colv.at[0]],
                                  buf.at[k], sem).wait()

    def _process(s, buf):
        for k in range(BATCH):
            j = s * BATCH + k
            bk = buf.at[k]

            def _scale(g, carry2, j=j, bk=bk):
                ew16 = ewv[pl.ds(j * CH + g * 16, 16)]
                for e in range(16):
                    splat = ew16.at[jnp.full((16,), e, jnp.int32)].get(
                        mode="promise_in_bounds")
                    r = g * 16 + e
                    w0 = bk[r, pl.ds(0, 16)]
                    w1 = bk[r, pl.ds(16, 16)]
                    lo0 = lax.bitcast_convert_type(jnp.left_shift(w0, 16), jnp.float32)
                    hi0 = lax.bitcast_convert_type(w0 & jnp.int32(-65536), jnp.float32)
                    lo1 = lax.bitcast_convert_type(jnp.left_shift(w1, 16), jnp.float32)
                    fbuf[r, pl.ds(0, 16)] = lo0 * splat
                    fbuf[r, pl.ds(16, 16)] = hi0 * splat
                    fbuf[r, pl.ds(32, 16)] = lo1 * splat
                return carry2

            lax.fori_loop(0, CH // 16, _scale, 0)
            pltpu.sync_copy(fbuf, acc.at[rowv.at[j]], add=True)

    _fire(0, bf_a, sem_a)

    def _pair(g, carry):
        s0 = 2 * g
        _drain(bf_a, sem_a)
        _fire(s0 + 1, bf_b, sem_b)
        _process(s0, bf_a)
        _drain(bf_b, sem_b)
        _fire(lax.rem(s0 + 2, NSUP), bf_a, sem_a)
        _process(s0 + 1, bf_b)
        return carry

    lax.fori_loop(0, NSUP // 2, _pair, 0)
    _drain(bf_a, sem_a)
    plsc.subcore_barrier()

    for i in range(STRIPE // CH):
        off = sid * STRIPE + i * CH
        pltpu.sync_copy(acc.at[pl.ds(off, CH)], fbuf)
        pltpu.sync_copy(fbuf, out_hbm.at[cid, pl.ds(off, CH)])


_sc_pass = pl.kernel(
    _sc_pass_body,
    out_type=jax.ShapeDtypeStruct((2, N_ACC, D), jnp.float32),
    mesh=plsc.VectorSubcoreMesh(core_axis_name="c", subcore_axis_name="s"),
    scratch_types=[
        pltpu.VMEM((NCH, CH), jnp.int32),
        pltpu.VMEM((NCH, CH), jnp.int32),
        pltpu.VMEM((EPW,), jnp.float32),
        pltpu.VMEM((BATCH, CH, DT), jnp.int32),
        pltpu.VMEM((BATCH, CH, DT), jnp.int32),
        pltpu.VMEM((CH, D), jnp.float32),
        pltpu.VMEM_SHARED((N_ACC, D), jnp.float32),
        pltpu.SemaphoreType.DMA,
        pltpu.SemaphoreType.DMA,
    ],
    compiler_params=pltpu.CompilerParams(use_tc_tiling_on_sc=False),
)


def _mid_body(acc_ref, t0_ref, taug_ref, dinv_ref):
    s = acc_ref[0] + acc_ref[1]
    deg = s[:, DEG_COL:DEG_COL + 1]
    dinv = 1.0 / jnp.where(deg > 0, deg, 1.0)
    taug_ref[...] = _to_table(t0_ref[...] + s * dinv)
    dinv_ref[...] = dinv


def _mid(acc1, t0):
    grid = (N // ROWBLK,)
    return pl.pallas_call(
        _mid_body,
        grid=grid,
        in_specs=[
            pl.BlockSpec((2, ROWBLK, D), lambda i: (0, i, 0)),
            pl.BlockSpec((ROWBLK, D), lambda i: (i, 0)),
        ],
        out_specs=[
            pl.BlockSpec((ROWBLK, DT), lambda i: (i, 0)),
            pl.BlockSpec((ROWBLK, 1), lambda i: (i, 0)),
        ],
        out_shape=[
            jax.ShapeDtypeStruct((N, DT), jnp.int32),
            jax.ShapeDtypeStruct((N, 1), jnp.float32),
        ],
    )(acc1, t0)


def _final_body(acc_ref, y0_ref, dinv_ref, out_ref):
    s = acc_ref[0] + acc_ref[1]
    out_ref[...] = y0_ref[:, :C_OUT] + s[:, :C_OUT] * dinv_ref[...]


def _final(acc2, y0b, dinv):
    grid = (N // ROWBLK,)
    return pl.pallas_call(
        _final_body,
        grid=grid,
        in_specs=[
            pl.BlockSpec((2, ROWBLK, D), lambda i: (0, i, 0)),
            pl.BlockSpec((ROWBLK, D), lambda i: (i, 0)),
            pl.BlockSpec((ROWBLK, 1), lambda i: (i, 0)),
        ],
        out_specs=pl.BlockSpec((ROWBLK, C_OUT), lambda i: (i, 0)),
        out_shape=jax.ShapeDtypeStruct((N, C_OUT), jnp.float32),
    )(acc2, y0b, dinv)


def kernel(x, edge_index, edge_weight, W, b, W2, b2):
    pad = (0, E_PAD - E)
    row3 = jnp.pad(edge_index[0], pad).reshape(NW, NCH, CH)
    col3 = jnp.pad(edge_index[1], pad).reshape(NW, NCH, CH)
    ew_p = jnp.pad(edge_weight, pad)
    b2d = b.reshape(1, UNITS)
    b22d = b2.reshape(1, C_OUT)

    y0b, t0, u_tab = _prep(x, W, b2d, W2, b22d)
    acc1 = _sc_pass(col3, row3, ew_p, u_tab)
    t_tab, dinv = _mid(acc1, t0)
    acc2 = _sc_pass(col3, row3, ew_p, t_tab)
    return _final(acc2, y0b, dinv)


# speedup vs baseline: 1.2340x; 1.0026x over previous
import jax
import jax.numpy as jnp
from jax import lax
from jax.experimental import pallas as pl
from jax.experimental.pallas import tpu as pltpu
from jax.experimental.pallas import tpu_sc as plsc

N = 10000
E = 320000
F = 128
UNITS = 64
C_OUT = 40
D = 48
DT = 32
DEG_COL = 40

NW = 32
CH = 128
NCH = 80
EPW = NCH * CH
E_PAD = NW * EPW
N_ACC = 10240
STRIPE = N_ACC // 16
ROWBLK = 1000
BATCH = 4
NSUP = NCH // BATCH


def _rnbf(x):
    b = lax.bitcast_convert_type(x, jnp.int32)
    rnd = b + jnp.int32(0x7FFF) + (lax.shift_right_arithmetic(b, 16) & 1)
    return lax.shift_right_logical(rnd, 16)


def _to_table(val48):
    g0 = jnp.left_shift(_rnbf(val48[:, 16:32]), 16) | _rnbf(val48[:, 0:16])
    g1 = _rnbf(val48[:, 32:48])
    return jnp.concatenate([g0, g1], axis=1)


def _prep_body(x_ref, w_ref, b_ref, w2_ref, b2_ref, y0_ref, t0_ref, u_ref):
    w_out = w2_ref[...]
    a0 = jnp.dot(w_ref[0] - w_ref[2], w_out, preferred_element_type=jnp.float32)
    a1 = -jnp.dot(w_ref[1], w_out, preferred_element_type=jnp.float32)
    a2 = 2.0 * jnp.dot(w_ref[2], w_out, preferred_element_type=jnp.float32)
    pad = jnp.zeros((F, D - C_OUT), jnp.float32)
    xb = x_ref[...]
    bias = jnp.dot(b_ref[...], w_out, preferred_element_type=jnp.float32) + b2_ref[...]
    biasp = jnp.concatenate([bias, jnp.zeros((1, D - C_OUT), jnp.float32)], axis=1)
    y0_ref[...] = jnp.dot(xb, jnp.concatenate([a0, pad], axis=1),
                          preferred_element_type=jnp.float32) + biasp
    t0_ref[...] = jnp.dot(xb, jnp.concatenate([a1, pad], axis=1),
                          preferred_element_type=jnp.float32)
    ids = lax.broadcasted_iota(jnp.int32, (1, D), 1)
    one_col = jnp.where(ids == DEG_COL, 1.0, 0.0)
    u48 = jnp.dot(xb, jnp.concatenate([a2, pad], axis=1),
                  preferred_element_type=jnp.float32) + one_col
    u_ref[...] = _to_table(u48)


def _prep(x, W, b2d, W2, b22d):
    grid = (N // ROWBLK,)
    return pl.pallas_call(
        _prep_body,
        grid=grid,
        in_specs=[
            pl.BlockSpec((ROWBLK, F), lambda i: (i, 0)),
            pl.BlockSpec((3, F, UNITS), lambda i: (0, 0, 0)),
            pl.BlockSpec((1, UNITS), lambda i: (0, 0)),
            pl.BlockSpec((UNITS, C_OUT), lambda i: (0, 0)),
            pl.BlockSpec((1, C_OUT), lambda i: (0, 0)),
        ],
        out_specs=[
            pl.BlockSpec((ROWBLK, D), lambda i: (i, 0)),
            pl.BlockSpec((ROWBLK, D), lambda i: (i, 0)),
            pl.BlockSpec((ROWBLK, DT), lambda i: (i, 0)),
        ],
        out_shape=[
            jax.ShapeDtypeStruct((N, D), jnp.float32),
            jax.ShapeDtypeStruct((N, D), jnp.float32),
            jax.ShapeDtypeStruct((N, DT), jnp.int32),
        ],
    )(x, W, b2d, W2, b22d)


def _sc_pass_body(col_hbm, row_hbm, ew_hbm, table_hbm, out_hbm,
                  colv, rowv, ewv, bf_a, bf_b, fbuf, acc, sem_a, sem_b):
    cid = lax.axis_index("c")
    sid = lax.axis_index("s")
    wid = sid * 2 + cid

    pltpu.sync_copy(col_hbm.at[wid], colv)
    pltpu.sync_copy(row_hbm.at[wid], rowv)
    pltpu.sync_copy(ew_hbm.at[pl.ds(wid * EPW, EPW)], ewv)

    zero16 = jnp.zeros((16,), jnp.float32)

    def _zrow(e, carry):
        for k in range(D // 16):
            fbuf[e, pl.ds(16 * k, 16)] = zero16
        return carry

    lax.fori_loop(0, CH, _zrow, 0)
    for i in range(STRIPE // CH):
        pltpu.sync_copy(fbuf, acc.at[pl.ds(sid * STRIPE + i * CH, CH)])
    plsc.subcore_barrier()

    def _fire(s, buf, sem):
        for k in range(BATCH):
            pltpu.async_copy(table_hbm.at[colv.at[s * BATCH + k]],
                             buf.at[k], sem)

    def _drain(buf, sem):
        for k in range(BATCH):
            pltpu.make_async_copy(table_hbm.at[colv.at[0]],
                                  buf.at[k], sem).wait()

    def _process(s, buf):
        for k in range(BATCH):
            j = s * BATCH + k
            bk = buf.at[k]

            def _scale(g, carry2, j=j, bk=bk):
                ew16 = ewv[pl.ds(j * CH + g * 16, 16)]
                for e in range(16):
                    splat = ew16.at[jnp.full((16,), e, jnp.int32)].get(
                        mode="promise_in_bounds")
                    r = g * 16 + e
                    w0 = bk[r, pl.ds(0, 16)]
                    w1 = bk[r, pl.ds(16, 16)]
                    lo0 = lax.bitcast_convert_type(jnp.left_shift(w0, 16), jnp.float32)
                    hi0 = lax.bitcast_convert_type(w0 & jnp.int32(-65536), jnp.float32)
                    lo1 = lax.bitcast_convert_type(jnp.left_shift(w1, 16), jnp.float32)
                    fbuf[r, pl.ds(0, 16)] = lo0 * splat
                    fbuf[r, pl.ds(16, 16)] = hi0 * splat
                    fbuf[r, pl.ds(32, 16)] = lo1 * splat
                return carry2

            lax.fori_loop(0, CH // 16, _scale, 0)
            pltpu.sync_copy(fbuf, acc.at[rowv.at[j]], add=True)

    _fire(0, bf_a, sem_a)

    def _pair(g, carry):
        s0 = 2 * g
        _drain(bf_a, sem_a)
        _fire(s0 + 1, bf_b, sem_b)
        _process(s0, bf_a)
        _drain(bf_b, sem_b)
        _fire(lax.rem(s0 + 2, NSUP), bf_a, sem_a)
        _process(s0 + 1, bf_b)
        return carry

    lax.fori_loop(0, NSUP // 2, _pair, 0)
    _drain(bf_a, sem_a)
    plsc.subcore_barrier()

    pltpu.sync_copy(acc.at[pl.ds(sid * STRIPE, STRIPE)],
                    out_hbm.at[cid, pl.ds(sid * STRIPE, STRIPE)])


_sc_pass = pl.kernel(
    _sc_pass_body,
    out_type=jax.ShapeDtypeStruct((2, N_ACC, D), jnp.float32),
    mesh=plsc.VectorSubcoreMesh(core_axis_name="c", subcore_axis_name="s"),
    scratch_types=[
        pltpu.VMEM((NCH, CH), jnp.int32),
        pltpu.VMEM((NCH, CH), jnp.int32),
        pltpu.VMEM((EPW,), jnp.float32),
        pltpu.VMEM((BATCH, CH, DT), jnp.int32),
        pltpu.VMEM((BATCH, CH, DT), jnp.int32),
        pltpu.VMEM((CH, D), jnp.float32),
        pltpu.VMEM_SHARED((N_ACC, D), jnp.float32),
        pltpu.SemaphoreType.DMA,
        pltpu.SemaphoreType.DMA,
    ],
    compiler_params=pltpu.CompilerParams(use_tc_tiling_on_sc=False),
)


def _mid_body(acc_ref, t0_ref, taug_ref, dinv_ref):
    s = acc_ref[0] + acc_ref[1]
    deg = s[:, DEG_COL:DEG_COL + 1]
    dinv = 1.0 / jnp.where(deg > 0, deg, 1.0)
    taug_ref[...] = _to_table(t0_ref[...] + s * dinv)
    dinv_ref[...] = dinv


def _mid(acc1, t0):
    grid = (N // ROWBLK,)
    return pl.pallas_call(
        _mid_body,
        grid=grid,
        in_specs=[
            pl.BlockSpec((2, ROWBLK, D), lambda i: (0, i, 0)),
            pl.BlockSpec((ROWBLK, D), lambda i: (i, 0)),
        ],
        out_specs=[
            pl.BlockSpec((ROWBLK, DT), lambda i: (i, 0)),
            pl.BlockSpec((ROWBLK, 1), lambda i: (i, 0)),
        ],
        out_shape=[
            jax.ShapeDtypeStruct((N, DT), jnp.int32),
            jax.ShapeDtypeStruct((N, 1), jnp.float32),
        ],
    )(acc1, t0)


def _final_body(acc_ref, y0_ref, dinv_ref, out_ref):
    s = acc_ref[0] + acc_ref[1]
    out_ref[...] = y0_ref[:, :C_OUT] + s[:, :C_OUT] * dinv_ref[...]


def _final(acc2, y0b, dinv):
    grid = (N // ROWBLK,)
    return pl.pallas_call(
        _final_body,
        grid=grid,
        in_specs=[
            pl.BlockSpec((2, ROWBLK, D), lambda i: (0, i, 0)),
            pl.BlockSpec((ROWBLK, D), lambda i: (i, 0)),
            pl.BlockSpec((ROWBLK, 1), lambda i: (i, 0)),
        ],
        out_specs=pl.BlockSpec((ROWBLK, C_OUT), lambda i: (i, 0)),
        out_shape=jax.ShapeDtypeStruct((N, C_OUT), jnp.float32),
    )(acc2, y0b, dinv)


def kernel(x, edge_index, edge_weight, W, b, W2, b2):
    pad = (0, E_PAD - E)
    row3 = jnp.pad(edge_index[0], pad).reshape(NW, NCH, CH)
    col3 = jnp.pad(edge_index[1], pad).reshape(NW, NCH, CH)
    ew_p = jnp.pad(edge_weight, pad)
    b2d = b.reshape(1, UNITS)
    b22d = b2.reshape(1, C_OUT)

    y0b, t0, u_tab = _prep(x, W, b2d, W2, b22d)
    acc1 = _sc_pass(col3, row3, ew_p, u_tab)
    t_tab, dinv = _mid(acc1, t0)
    acc2 = _sc_pass(col3, row3, ew_p, t_tab)
    return _final(acc2, y0b, dinv)


# speedup vs baseline: 1.2780x; 1.0357x over previous
import jax
import jax.numpy as jnp
from jax import lax
from jax.experimental import pallas as pl
from jax.experimental.pallas import tpu as pltpu
from jax.experimental.pallas import tpu_sc as plsc

N = 10000
E = 320000
F = 128
UNITS = 64
C_OUT = 40
D = 48
DT = 32
DEG_COL = 40

NW = 32
CH = 128
NCH = 80
EPW = NCH * CH
E_PAD = NW * EPW
N_ACC = 10240
STRIPE = N_ACC // 16
ROWBLK = 1000
BATCH = 4
NSUP = NCH // BATCH


def _rnbf(x):
    b = lax.bitcast_convert_type(x, jnp.int32)
    rnd = b + jnp.int32(0x7FFF) + (lax.shift_right_arithmetic(b, 16) & 1)
    return lax.shift_right_logical(rnd, 16)


def _to_table(val48):
    g0 = jnp.left_shift(_rnbf(val48[:, 16:32]), 16) | _rnbf(val48[:, 0:16])
    g1 = _rnbf(val48[:, 32:48])
    return jnp.concatenate([g0, g1], axis=1)


def _prep_u_body(x_ref, w_ref, w2_ref, u_ref):
    a2 = 2.0 * jnp.dot(w_ref[2], w2_ref[...], preferred_element_type=jnp.float32)
    pad = jnp.zeros((F, D - C_OUT), jnp.float32)
    ids = lax.broadcasted_iota(jnp.int32, (1, D), 1)
    one_col = jnp.where(ids == DEG_COL, 1.0, 0.0)
    u48 = jnp.dot(x_ref[...], jnp.concatenate([a2, pad], axis=1),
                  preferred_element_type=jnp.float32) + one_col
    u_ref[...] = _to_table(u48)


def _prep_u(x, W, W2):
    grid = (N // ROWBLK,)
    return pl.pallas_call(
        _prep_u_body,
        grid=grid,
        in_specs=[
            pl.BlockSpec((ROWBLK, F), lambda i: (i, 0)),
            pl.BlockSpec((3, F, UNITS), lambda i: (0, 0, 0)),
            pl.BlockSpec((UNITS, C_OUT), lambda i: (0, 0)),
        ],
        out_specs=pl.BlockSpec((ROWBLK, DT), lambda i: (i, 0)),
        out_shape=jax.ShapeDtypeStruct((N, DT), jnp.int32),
    )(x, W, W2)


def _prep_yt_body(x_ref, w_ref, b_ref, w2_ref, b2_ref, y0_ref, t0_ref):
    w_out = w2_ref[...]
    a0 = jnp.dot(w_ref[0] - w_ref[2], w_out, preferred_element_type=jnp.float32)
    a1 = -jnp.dot(w_ref[1], w_out, preferred_element_type=jnp.float32)
    pad = jnp.zeros((F, D - C_OUT), jnp.float32)
    xb = x_ref[...]
    bias = jnp.dot(b_ref[...], w_out, preferred_element_type=jnp.float32) + b2_ref[...]
    biasp = jnp.concatenate([bias, jnp.zeros((1, D - C_OUT), jnp.float32)], axis=1)
    y0_ref[...] = jnp.dot(xb, jnp.concatenate([a0, pad], axis=1),
                          preferred_element_type=jnp.float32) + biasp
    t0_ref[...] = jnp.dot(xb, jnp.concatenate([a1, pad], axis=1),
                          preferred_element_type=jnp.float32)


def _prep_yt(x, W, b2d, W2, b22d):
    grid = (N // ROWBLK,)
    return pl.pallas_call(
        _prep_yt_body,
        grid=grid,
        in_specs=[
            pl.BlockSpec((ROWBLK, F), lambda i: (i, 0)),
            pl.BlockSpec((3, F, UNITS), lambda i: (0, 0, 0)),
            pl.BlockSpec((1, UNITS), lambda i: (0, 0)),
            pl.BlockSpec((UNITS, C_OUT), lambda i: (0, 0)),
            pl.BlockSpec((1, C_OUT), lambda i: (0, 0)),
        ],
        out_specs=[
            pl.BlockSpec((ROWBLK, D), lambda i: (i, 0)),
            pl.BlockSpec((ROWBLK, D), lambda i: (i, 0)),
        ],
        out_shape=[
            jax.ShapeDtypeStruct((N, D), jnp.float32),
            jax.ShapeDtypeStruct((N, D), jnp.float32),
        ],
    )(x, W, b2d, W2, b22d)


def _sc_pass_body(col_hbm, row_hbm, ew_hbm, table_hbm, out_hbm,
                  colv, rowv, ewv, bf_a, bf_b, fbuf, acc, sem_a, sem_b):
    cid = lax.axis_index("c")
    sid = lax.axis_index("s")
    wid = sid * 2 + cid

    pltpu.sync_copy(col_hbm.at[wid], colv)
    pltpu.sync_copy(row_hbm.at[wid], rowv)
    pltpu.sync_copy(ew_hbm.at[pl.ds(wid * EPW, EPW)], ewv)

    zero16 = jnp.zeros((16,), jnp.float32)

    def _zrow(e, carry):
        for k in range(D // 16):
            fbuf[e, pl.ds(16 * k, 16)] = zero16
        return carry

    lax.fori_loop(0, CH, _zrow, 0)
    for i in range(STRIPE // CH):
        pltpu.sync_copy(fbuf, acc.at[pl.ds(sid * STRIPE + i * CH, CH)])
    plsc.subcore_barrier()

    def _fire(s, buf, sem):
        for k in range(BATCH):
            pltpu.async_copy(table_hbm.at[colv.at[s * BATCH + k]],
                             buf.at[k], sem)

    def _drain(buf, sem):
        for k in range(BATCH):
            pltpu.make_async_copy(table_hbm.at[colv.at[0]],
                                  buf.at[k], sem).wait()

    def _process(s, buf):
        for k in range(BATCH):
            j = s * BATCH + k
            bk = buf.at[k]

            def _scale(g, carry2, j=j, bk=bk):
                ew16 = ewv[pl.ds(j * CH + g * 16, 16)]
                for e in range(16):
                    splat = ew16.at[jnp.full((16,), e, jnp.int32)].get(
                        mode="promise_in_bounds")
                    r = g * 16 + e
                    w0 = bk[r, pl.ds(0, 16)]
                    w1 = bk[r, pl.ds(16, 16)]
                    lo0 = lax.bitcast_convert_type(jnp.left_shift(w0, 16), jnp.float32)
                    hi0 = lax.bitcast_convert_type(w0 & jnp.int32(-65536), jnp.float32)
                    lo1 = lax.bitcast_convert_type(jnp.left_shift(w1, 16), jnp.float32)
                    fbuf[r, pl.ds(0, 16)] = lo0 * splat
                    fbuf[r, pl.ds(16, 16)] = hi0 * splat
                    fbuf[r, pl.ds(32, 16)] = lo1 * splat
                return carry2

            lax.fori_loop(0, CH // 16, _scale, 0)
            pltpu.sync_copy(fbuf, acc.at[rowv.at[j]], add=True)

    _fire(0, bf_a, sem_a)

    def _pair(g, carry):
        s0 = 2 * g
        _drain(bf_a, sem_a)
        _fire(s0 + 1, bf_b, sem_b)
        _process(s0, bf_a)
        _drain(bf_b, sem_b)
        _fire(lax.rem(s0 + 2, NSUP), bf_a, sem_a)
        _process(s0 + 1, bf_b)
        return carry

    lax.fori_loop(0, NSUP // 2, _pair, 0)
    _drain(bf_a, sem_a)
    plsc.subcore_barrier()

    pltpu.sync_copy(acc.at[pl.ds(sid * STRIPE, STRIPE)],
                    out_hbm.at[cid, pl.ds(sid * STRIPE, STRIPE)])


_sc_pass = pl.kernel(
    _sc_pass_body,
    out_type=jax.ShapeDtypeStruct((2, N_ACC, D), jnp.float32),
    mesh=plsc.VectorSubcoreMesh(core_axis_name="c", subcore_axis_name="s"),
    scratch_types=[
        pltpu.VMEM((NCH, CH), jnp.int32),
        pltpu.VMEM((NCH, CH), jnp.int32),
        pltpu.VMEM((EPW,), jnp.float32),
        pltpu.VMEM((BATCH, CH, DT), jnp.int32),
        pltpu.VMEM((BATCH, CH, DT), jnp.int32),
        pltpu.VMEM((CH, D), jnp.float32),
        pltpu.VMEM_SHARED((N_ACC, D), jnp.float32),
        pltpu.SemaphoreType.DMA,
        pltpu.SemaphoreType.DMA,
    ],
    compiler_params=pltpu.CompilerParams(use_tc_tiling_on_sc=False),
)


def _mid_body(acc_ref, t0_ref, taug_ref, dinv_ref):
    s = acc_ref[0] + acc_ref[1]
    deg = s[:, DEG_COL:DEG_COL + 1]
    dinv = 1.0 / jnp.where(deg > 0, deg, 1.0)
    taug_ref[...] = _to_table(t0_ref[...] + s * dinv)
    dinv_ref[...] = dinv


def _mid(acc1, t0):
    grid = (N // ROWBLK,)
    return pl.pallas_call(
        _mid_body,
        grid=grid,
        in_specs=[
            pl.BlockSpec((2, ROWBLK, D), lambda i: (0, i, 0)),
            pl.BlockSpec((ROWBLK, D), lambda i: (i, 0)),
        ],
        out_specs=[
            pl.BlockSpec((ROWBLK, DT), lambda i: (i, 0)),
            pl.BlockSpec((ROWBLK, 1), lambda i: (i, 0)),
        ],
        out_shape=[
            jax.ShapeDtypeStruct((N, DT), jnp.int32),
            jax.ShapeDtypeStruct((N, 1), jnp.float32),
        ],
    )(acc1, t0)


def _final_body(acc_ref, y0_ref, dinv_ref, out_ref):
    s = acc_ref[0] + acc_ref[1]
    out_ref[...] = y0_ref[:, :C_OUT] + s[:, :C_OUT] * dinv_ref[...]


def _final(acc2, y0b, dinv):
    grid = (N // ROWBLK,)
    return pl.pallas_call(
        _final_body,
        grid=grid,
        in_specs=[
            pl.BlockSpec((2, ROWBLK, D), lambda i: (0, i, 0)),
            pl.BlockSpec((ROWBLK, D), lambda i: (i, 0)),
            pl.BlockSpec((ROWBLK, 1), lambda i: (i, 0)),
        ],
        out_specs=pl.BlockSpec((ROWBLK, C_OUT), lambda i: (i, 0)),
        out_shape=jax.ShapeDtypeStruct((N, C_OUT), jnp.float32),
    )(acc2, y0b, dinv)


def kernel(x, edge_index, edge_weight, W, b, W2, b2):
    pad = (0, E_PAD - E)
    row3 = jnp.pad(edge_index[0], pad).reshape(NW, NCH, CH)
    col3 = jnp.pad(edge_index[1], pad).reshape(NW, NCH, CH)
    ew_p = jnp.pad(edge_weight, pad)
    b2d = b.reshape(1, UNITS)
    b22d = b2.reshape(1, C_OUT)

    u_tab = _prep_u(x, W, W2)
    acc1 = _sc_pass(col3, row3, ew_p, u_tab)
    y0b, t0 = _prep_yt(x, W, b2d, W2, b22d)
    t_tab, dinv = _mid(acc1, t0)
    acc2 = _sc_pass(col3, row3, ew_p, t_tab)
    return _final(acc2, y0b, dinv)
